# Initial kernel scaffold; baseline (speedup 1.0000x reference)
import jax
import jax.numpy as jnp
from jax.experimental import pallas as pl


def kernel(boxes, scores, labels):
    raise NotImplementedError("write your pallas kernel here")


# speedup vs baseline: 51.1453x; 51.1453x over previous
import functools

import jax
import jax.numpy as jnp
from jax import lax
from jax.experimental import pallas as pl
from jax.experimental.pallas import tpu as pltpu
from jax.experimental.pallas import tpu_sc as plsc

_N = 5000
_NPAD = 5008
_NKEY = 5120
_ROWS = _NKEY // 128
_SCORE_THRESH = 0.2
_NMS_THRESH = 0.5
_MAXK = 15


def _rank_body(score_ref, rank_ref):
    s = score_ref[...]
    keys = jnp.where(s >= _SCORE_THRESH, s, jnp.float32(-1.0))
    ii = (lax.broadcasted_iota(jnp.int32, (_ROWS, 128), 0) * 128
          + lax.broadcasted_iota(jnp.int32, (_ROWS, 128), 1))

    def body(cj, acc):
        sj = score_ref[pl.ds(cj, 1), :]
        kj = jnp.where(sj >= _SCORE_THRESH, sj,
                       jnp.float32(-1.0)).reshape(1, 1, 128)
        jj = cj * 128 + lax.broadcasted_iota(jnp.int32, (1, 1, 128), 2)
        before = (kj > keys[:, :, None]) | (
            (kj == keys[:, :, None]) & (jj < ii[:, :, None]))
        return acc + jnp.sum(before.astype(jnp.int32), axis=2)

    rank_ref[...] = lax.fori_loop(0, _ROWS, body,
                                  jnp.zeros((_ROWS, 128), jnp.int32))


def _sc_body(boxes_hbm, scores_hbm, labels_hbm, rank_hbm, out_hbm,
             bx_v, sc_v, lb_v, rk_v,
             sox0, soy0, sox1, soy1, sarea, ssc, slab,
             srx0, sry0, srx1, sry1, out_v):
    f32 = jnp.float32
    i32 = jnp.int32
    cid = lax.axis_index("c")
    sid = lax.axis_index("s")

    @pl.when((cid == 0) & (sid == 0))
    def _work():
        pltpu.sync_copy(boxes_hbm, bx_v)
        pltpu.sync_copy(scores_hbm, sc_v)
        pltpu.sync_copy(labels_hbm, lb_v)
        pltpu.sync_copy(rank_hbm, rk_v)

        def mbody(k, m):
            return jnp.maximum(m, bx_v[pl.ds(k * 16, 16)])
        mvec = lax.fori_loop(0, (_NPAD * 4) // 16, mbody,
                             jnp.full((16,), -3e38, f32))
        m1 = jnp.max(mvec) + f32(1.0)

        lanes = lax.iota(i32, 16)

        def abody(k, _):
            idx = k * 16 + lanes
            r = rk_v[pl.ds(k * 16, 16)]
            s = sc_v[pl.ds(k * 16, 16)]
            l = lb_v[pl.ds(k * 16, 16)]
            x0 = plsc.load_gather(bx_v, [idx * 4])
            y0 = plsc.load_gather(bx_v, [idx * 4 + 1])
            x1 = plsc.load_gather(bx_v, [idx * 4 + 2])
            y1 = plsc.load_gather(bx_v, [idx * 4 + 3])
            off = l.astype(f32) * m1
            ox0 = x0 + off
            oy0 = y0 + off
            ox1 = x1 + off
            oy1 = y1 + off
            ar = (ox1 - ox0) * (oy1 - oy0)
            msk = idx < _N
            plsc.store_scatter(sox0, [r], ox0, mask=msk)
            plsc.store_scatter(soy0, [r], oy0, mask=msk)
            plsc.store_scatter(sox1, [r], ox1, mask=msk)
            plsc.store_scatter(soy1, [r], oy1, mask=msk)
            plsc.store_scatter(sarea, [r], ar, mask=msk)
            plsc.store_scatter(ssc, [r], s, mask=msk)
            plsc.store_scatter(slab, [r], l, mask=msk)
            plsc.store_scatter(srx0, [r], x0, mask=msk)
            plsc.store_scatter(sry0, [r], y0, mask=msk)
            plsc.store_scatter(srx1, [r], x1, mask=msk)
            plsc.store_scatter(sry1, [r], y1, mask=msk)
            return 0
        lax.fori_loop(0, _NPAD // 16, abody, 0)

        LO = f32(1e30)
        HI = f32(-1e30)
        zf = jnp.zeros((16,), f32)
        init = (
            i32(0), i32(0), i32(0), False,
            jnp.full((16,), LO, f32), jnp.full((16,), LO, f32),
            jnp.full((16,), HI, f32), jnp.full((16,), HI, f32), zf,
            zf, zf, zf, zf, zf,
            jnp.full((16,), LO, f32), jnp.full((16,), LO, f32),
            jnp.full((16,), HI, f32), jnp.full((16,), HI, f32), zf,
            zf, zf, zf, zf, zf,
        )

        def cond(carry):
            t, hc, oc, stop = carry[0], carry[1], carry[2], carry[3]
            return (~stop) & (t < _N)

        def body(carry):
            (t, hc, oc, stop,
             hox0, hoy0, hox1, hoy1, har, hrx0, hry0, hrx1, hry1, hsc,
             qox0, qoy0, qox1, qoy1, qar, qrx0, qry0, qrx1, qry1, qsc) = carry
            tv = jnp.full((16,), t, i32)
            s = plsc.load_gather(ssc, [tv])
            l = plsc.load_gather(slab, [tv])
            cox0 = plsc.load_gather(sox0, [tv])
            coy0 = plsc.load_gather(soy0, [tv])
            cox1 = plsc.load_gather(sox1, [tv])
            coy1 = plsc.load_gather(soy1, [tv])
            car = plsc.load_gather(sarea, [tv])
            s_sc = jnp.max(s)
            is_h = jnp.max(l) == 0
            invalid = s_sc < f32(_SCORE_THRESH)

            kox0 = jnp.where(is_h, hox0, qox0)
            koy0 = jnp.where(is_h, hoy0, qoy0)
            kox1 = jnp.where(is_h, hox1, qox1)
            koy1 = jnp.where(is_h, hoy1, qoy1)
            kar = jnp.where(is_h, har, qar)
            lt0 = jnp.maximum(cox0, kox0)
            lt1 = jnp.maximum(coy0, koy0)
            rb0 = jnp.minimum(cox1, kox1)
            rb1 = jnp.minimum(coy1, koy1)
            w = jnp.maximum(rb0 - lt0, f32(0.0))
            h = jnp.maximum(rb1 - lt1, f32(0.0))
            inter = w * h
            union = car + kar - inter
            iou = inter / jnp.maximum(union, f32(1e-9))
            suppressed = jnp.any(iou > f32(_NMS_THRESH))

            cnt = jnp.where(is_h, hc, oc)
            do_app = (~invalid) & (~suppressed) & (cnt < _MAXK)
            amask = (lanes == cnt) & do_app
            ah = amask & is_h
            ao = amask & (~is_h)

            crx0 = plsc.load_gather(srx0, [tv])
            cry0 = plsc.load_gather(sry0, [tv])
            crx1 = plsc.load_gather(srx1, [tv])
            cry1 = plsc.load_gather(sry1, [tv])

            hox0 = jnp.where(ah, cox0, hox0)
            hoy0 = jnp.where(ah, coy0, hoy0)
            hox1 = jnp.where(ah, cox1, hox1)
            hoy1 = jnp.where(ah, coy1, hoy1)
            har = jnp.where(ah, car, har)
            hrx0 = jnp.where(ah, crx0, hrx0)
            hry0 = jnp.where(ah, cry0, hry0)
            hrx1 = jnp.where(ah, crx1, hrx1)
            hry1 = jnp.where(ah, cry1, hry1)
            hsc = jnp.where(ah, s, hsc)
            qox0 = jnp.where(ao, cox0, qox0)
            qoy0 = jnp.where(ao, coy0, qoy0)
            qox1 = jnp.where(ao, cox1, qox1)
            qoy1 = jnp.where(ao, coy1, qoy1)
            qar = jnp.where(ao, car, qar)
            qrx0 = jnp.where(ao, crx0, qrx0)
            qry0 = jnp.where(ao, cry0, qry0)
            qrx1 = jnp.where(ao, crx1, qrx1)
            qry1 = jnp.where(ao, cry1, qry1)
            qsc = jnp.where(ao, s, qsc)

            inc = jnp.where(do_app, i32(1), i32(0))
            hc = hc + jnp.where(is_h, inc, i32(0))
            oc = oc + jnp.where(is_h, i32(0), inc)
            stop = invalid | ((hc >= _MAXK) & (oc >= _MAXK))
            return (t + 1, hc, oc, stop,
                    hox0, hoy0, hox1, hoy1, har, hrx0, hry0, hrx1, hry1, hsc,
                    qox0, qoy0, qox1, qoy1, qar, qrx0, qry0, qrx1, qry1, qsc)

        fin = lax.while_loop(cond, body, init)
        (hrx0, hry0, hrx1, hry1, hsc) = fin[9:14]
        (qrx0, qry0, qrx1, qry1, qsc) = fin[19:24]

        m15 = lanes < _MAXK
        cols = [hrx0, hry0, hrx1, hry1, hsc]
        colsq = [qrx0, qry0, qrx1, qry1, qsc]
        for c in range(5):
            cv = jnp.full((16,), c, i32)
            plsc.store_scatter(out_v, [lanes, cv], cols[c], mask=m15)
            plsc.store_scatter(out_v, [lanes + _MAXK, cv], colsq[c], mask=m15)
        pltpu.sync_copy(out_v, out_hbm)


@functools.partial(jax.jit, static_argnums=())
def kernel(boxes, scores, labels):
    f32 = jnp.float32
    i32 = jnp.int32

    scores_p = jnp.pad(scores, (0, _NKEY - _N),
                       constant_values=jnp.float32(-1000.0)).reshape(_ROWS, 128)
    rank2d = pl.pallas_call(
        _rank_body,
        out_shape=jax.ShapeDtypeStruct((_ROWS, 128), i32),
    )(scores_p)
    rank = rank2d.reshape(_NKEY)[:_NPAD]

    boxes_f = jnp.pad(boxes.reshape(-1), (0, (_NPAD - _N) * 4))
    scores_f = jnp.pad(scores, (0, _NPAD - _N))
    labels_f = jnp.pad(labels, (0, _NPAD - _N))

    sc_fn = functools.partial(
        pl.kernel,
        out_type=jax.ShapeDtypeStruct((2 * _MAXK, 5), f32),
        mesh=plsc.VectorSubcoreMesh(core_axis_name="c", subcore_axis_name="s"),
        compiler_params=pltpu.CompilerParams(needs_layout_passes=False),
        scratch_types=[
            pltpu.VMEM((_NPAD * 4,), f32),
            pltpu.VMEM((_NPAD,), f32),
            pltpu.VMEM((_NPAD,), i32),
            pltpu.VMEM((_NPAD,), i32),
            pltpu.VMEM((_NPAD,), f32),
            pltpu.VMEM((_NPAD,), f32),
            pltpu.VMEM((_NPAD,), f32),
            pltpu.VMEM((_NPAD,), f32),
            pltpu.VMEM((_NPAD,), f32),
            pltpu.VMEM((_NPAD,), f32),
            pltpu.VMEM((_NPAD,), i32),
            pltpu.VMEM((_NPAD,), f32),
            pltpu.VMEM((_NPAD,), f32),
            pltpu.VMEM((_NPAD,), f32),
            pltpu.VMEM((_NPAD,), f32),
            pltpu.VMEM((2 * _MAXK, 5), f32),
        ],
    )(_sc_body)
    return sc_fn(boxes_f, scores_f, labels_f, rank)


# speedup vs baseline: 92.7780x; 1.8140x over previous
import functools

import jax
import jax.numpy as jnp
from jax import lax
from jax.experimental import pallas as pl
from jax.experimental.pallas import tpu as pltpu
from jax.experimental.pallas import tpu_sc as plsc

_N = 5000
_NPAD = 5008
_NKEY = 5120
_ROWS = _NKEY // 128
_SCORE_THRESH = 0.2
_NMS_THRESH = 0.5
_MAXK = 15


def _rank_body(srow_ref, scol_ref, bx_ref, rank_ref, maxc_ref):
    f32 = jnp.float32
    i32 = jnp.int32
    neg1 = f32(-1.0)
    thr = f32(_SCORE_THRESH)
    tri = (lax.broadcasted_iota(i32, (128, 128), 0)
           < lax.broadcasted_iota(i32, (128, 128), 1))

    for c in range(_ROWS):
        s_i = srow_ref[pl.ds(c, 1), :]
        ki = jnp.where(s_i >= thr, s_i, neg1)
        cnt = jnp.zeros((128,), i32)
        if c > 0:
            topv = scol_ref[pl.ds(0, c * 128), :]
            ktop = jnp.where(topv >= thr, topv, neg1)
            cnt = cnt + jnp.sum((ktop >= ki).astype(i32), axis=0)
        if c < _ROWS - 1:
            botv = scol_ref[pl.ds((c + 1) * 128, (_ROWS - 1 - c) * 128), :]
            kbot = jnp.where(botv >= thr, botv, neg1)
            cnt = cnt + jnp.sum((kbot > ki).astype(i32), axis=0)
        bandv = scol_ref[pl.ds(c * 128, 128), :]
        kband = jnp.where(bandv >= thr, bandv, neg1)
        tb = (kband > ki) | ((kband == ki) & tri)
        cnt = cnt + jnp.sum(tb.astype(i32), axis=0)
        rank_ref[pl.ds(c, 1), :] = cnt.reshape(1, 128)

    maxc_ref[...] = jnp.full((1, 128), jnp.max(bx_ref[...]), f32)


def _sc_body(boxes_hbm, scores_hbm, labels_hbm, rank_hbm, maxc_hbm, out_hbm,
             bx_v, sc_v, lb_v, rk_v, mx_v,
             sox0, soy0, sox1, soy1, sarea, ssc, slab,
             srx0, sry0, srx1, sry1, out_v):
    f32 = jnp.float32
    i32 = jnp.int32
    cid = lax.axis_index("c")
    sid = lax.axis_index("s")

    @pl.when((cid == 0) & (sid == 0))
    def _work():
        pltpu.sync_copy(boxes_hbm, bx_v)
        pltpu.sync_copy(scores_hbm, sc_v)
        pltpu.sync_copy(labels_hbm, lb_v)
        pltpu.sync_copy(rank_hbm, rk_v)
        pltpu.sync_copy(maxc_hbm, mx_v)

        m1 = lax.slice(mx_v[pl.ds(0, 16)], (0,), (1,))[0] + f32(1.0)

        lanes = lax.iota(i32, 16)

        def abody(k, _):
            idx = k * 16 + lanes
            r = rk_v[pl.ds(k * 16, 16)]
            s = sc_v[pl.ds(k * 16, 16)]
            l = lb_v[pl.ds(k * 16, 16)]
            x0 = plsc.load_gather(bx_v, [idx * 4])
            y0 = plsc.load_gather(bx_v, [idx * 4 + 1])
            x1 = plsc.load_gather(bx_v, [idx * 4 + 2])
            y1 = plsc.load_gather(bx_v, [idx * 4 + 3])
            off = l.astype(f32) * m1
            ox0 = x0 + off
            oy0 = y0 + off
            ox1 = x1 + off
            oy1 = y1 + off
            ar = (ox1 - ox0) * (oy1 - oy0)
            msk = idx < _N
            plsc.store_scatter(sox0, [r], ox0, mask=msk)
            plsc.store_scatter(soy0, [r], oy0, mask=msk)
            plsc.store_scatter(sox1, [r], ox1, mask=msk)
            plsc.store_scatter(soy1, [r], oy1, mask=msk)
            plsc.store_scatter(sarea, [r], ar, mask=msk)
            plsc.store_scatter(ssc, [r], s, mask=msk)
            plsc.store_scatter(slab, [r], l, mask=msk)
            plsc.store_scatter(srx0, [r], x0, mask=msk)
            plsc.store_scatter(sry0, [r], y0, mask=msk)
            plsc.store_scatter(srx1, [r], x1, mask=msk)
            plsc.store_scatter(sry1, [r], y1, mask=msk)
            return 0
        lax.fori_loop(0, _NPAD // 16, abody, 0)

        LO = f32(1e30)
        HI = f32(-1e30)
        zf = jnp.zeros((16,), f32)
        init = (
            i32(0), i32(0), i32(0), False,
            jnp.full((16,), LO, f32), jnp.full((16,), LO, f32),
            jnp.full((16,), HI, f32), jnp.full((16,), HI, f32), zf,
            zf, zf, zf, zf, zf,
            jnp.full((16,), LO, f32), jnp.full((16,), LO, f32),
            jnp.full((16,), HI, f32), jnp.full((16,), HI, f32), zf,
            zf, zf, zf, zf, zf,
        )

        def cond(carry):
            t, hc, oc, stop = carry[0], carry[1], carry[2], carry[3]
            return (~stop) & (t < _N)

        def body(carry):
            (t, hc, oc, stop,
             hox0, hoy0, hox1, hoy1, har, hrx0, hry0, hrx1, hry1, hsc,
             qox0, qoy0, qox1, qoy1, qar, qrx0, qry0, qrx1, qry1, qsc) = carry
            tv = jnp.full((16,), t, i32)
            s = plsc.load_gather(ssc, [tv])
            l = plsc.load_gather(slab, [tv])
            cox0 = plsc.load_gather(sox0, [tv])
            coy0 = plsc.load_gather(soy0, [tv])
            cox1 = plsc.load_gather(sox1, [tv])
            coy1 = plsc.load_gather(soy1, [tv])
            car = plsc.load_gather(sarea, [tv])
            s_sc = lax.slice(s, (0,), (1,))[0]
            is_h = lax.slice(l, (0,), (1,))[0] == 0
            invalid = s_sc < f32(_SCORE_THRESH)

            kox0 = jnp.where(is_h, hox0, qox0)
            koy0 = jnp.where(is_h, hoy0, qoy0)
            kox1 = jnp.where(is_h, hox1, qox1)
            koy1 = jnp.where(is_h, hoy1, qoy1)
            kar = jnp.where(is_h, har, qar)
            lt0 = jnp.maximum(cox0, kox0)
            lt1 = jnp.maximum(coy0, koy0)
            rb0 = jnp.minimum(cox1, kox1)
            rb1 = jnp.minimum(coy1, koy1)
            w = jnp.maximum(rb0 - lt0, f32(0.0))
            h = jnp.maximum(rb1 - lt1, f32(0.0))
            inter = w * h
            union = car + kar - inter
            iou = inter / jnp.maximum(union, f32(1e-9))
            suppressed = jnp.any(iou > f32(_NMS_THRESH))

            cnt = jnp.where(is_h, hc, oc)
            do_app = (~invalid) & (~suppressed) & (cnt < _MAXK)
            amask = (lanes == cnt) & do_app
            ah = amask & is_h
            ao = amask & (~is_h)

            crx0 = plsc.load_gather(srx0, [tv])
            cry0 = plsc.load_gather(sry0, [tv])
            crx1 = plsc.load_gather(srx1, [tv])
            cry1 = plsc.load_gather(sry1, [tv])

            hox0 = jnp.where(ah, cox0, hox0)
            hoy0 = jnp.where(ah, coy0, hoy0)
            hox1 = jnp.where(ah, cox1, hox1)
            hoy1 = jnp.where(ah, coy1, hoy1)
            har = jnp.where(ah, car, har)
            hrx0 = jnp.where(ah, crx0, hrx0)
            hry0 = jnp.where(ah, cry0, hry0)
            hrx1 = jnp.where(ah, crx1, hrx1)
            hry1 = jnp.where(ah, cry1, hry1)
            hsc = jnp.where(ah, s, hsc)
            qox0 = jnp.where(ao, cox0, qox0)
            qoy0 = jnp.where(ao, coy0, qoy0)
            qox1 = jnp.where(ao, cox1, qox1)
            qoy1 = jnp.where(ao, coy1, qoy1)
            qar = jnp.where(ao, car, qar)
            qrx0 = jnp.where(ao, crx0, qrx0)
            qry0 = jnp.where(ao, cry0, qry0)
            qrx1 = jnp.where(ao, crx1, qrx1)
            qry1 = jnp.where(ao, cry1, qry1)
            qsc = jnp.where(ao, s, qsc)

            inc = jnp.where(do_app, i32(1), i32(0))
            hc = hc + jnp.where(is_h, inc, i32(0))
            oc = oc + jnp.where(is_h, i32(0), inc)
            stop = invalid | ((hc >= _MAXK) & (oc >= _MAXK))
            return (t + 1, hc, oc, stop,
                    hox0, hoy0, hox1, hoy1, har, hrx0, hry0, hrx1, hry1, hsc,
                    qox0, qoy0, qox1, qoy1, qar, qrx0, qry0, qrx1, qry1, qsc)

        fin = lax.while_loop(cond, body, init)
        (hrx0, hry0, hrx1, hry1, hsc) = fin[9:14]
        (qrx0, qry0, qrx1, qry1, qsc) = fin[19:24]

        m15 = lanes < _MAXK
        cols = [hrx0, hry0, hrx1, hry1, hsc]
        colsq = [qrx0, qry0, qrx1, qry1, qsc]
        for c in range(5):
            cv = jnp.full((16,), c, i32)
            plsc.store_scatter(out_v, [lanes, cv], cols[c], mask=m15)
            plsc.store_scatter(out_v, [lanes + _MAXK, cv], colsq[c], mask=m15)
        pltpu.sync_copy(out_v, out_hbm)


@functools.partial(jax.jit, static_argnums=())
def kernel(boxes, scores, labels):
    f32 = jnp.float32
    i32 = jnp.int32

    scores_p = jnp.pad(scores, (0, _NKEY - _N),
                       constant_values=jnp.float32(-1000.0))
    srow = scores_p.reshape(_ROWS, 128)
    scol = scores_p.reshape(_NKEY, 1)
    boxes_2d = jnp.pad(boxes.reshape(-1), (0, 20480 - 4 * _N)).reshape(160, 128)
    rank2d, maxc = pl.pallas_call(
        _rank_body,
        out_shape=(jax.ShapeDtypeStruct((_ROWS, 128), i32),
                   jax.ShapeDtypeStruct((1, 128), f32)),
    )(srow, scol, boxes_2d)
    rank = rank2d.reshape(_NKEY)[:_NPAD]
    maxc_f = maxc.reshape(128)

    boxes_f = jnp.pad(boxes.reshape(-1), (0, (_NPAD - _N) * 4))
    scores_f = jnp.pad(scores, (0, _NPAD - _N))
    labels_f = jnp.pad(labels, (0, _NPAD - _N))

    sc_fn = functools.partial(
        pl.kernel,
        out_type=jax.ShapeDtypeStruct((2 * _MAXK, 5), f32),
        mesh=plsc.VectorSubcoreMesh(core_axis_name="c", subcore_axis_name="s"),
        compiler_params=pltpu.CompilerParams(needs_layout_passes=False),
        scratch_types=[
            pltpu.VMEM((_NPAD * 4,), f32),
            pltpu.VMEM((_NPAD,), f32),
            pltpu.VMEM((_NPAD,), i32),
            pltpu.VMEM((_NPAD,), i32),
            pltpu.VMEM((128,), f32),
            pltpu.VMEM((_NPAD,), f32),
            pltpu.VMEM((_NPAD,), f32),
            pltpu.VMEM((_NPAD,), f32),
            pltpu.VMEM((_NPAD,), f32),
            pltpu.VMEM((_NPAD,), f32),
            pltpu.VMEM((_NPAD,), f32),
            pltpu.VMEM((_NPAD,), i32),
            pltpu.VMEM((_NPAD,), f32),
            pltpu.VMEM((_NPAD,), f32),
            pltpu.VMEM((_NPAD,), f32),
            pltpu.VMEM((_NPAD,), f32),
            pltpu.VMEM((2 * _MAXK, 5), f32),
        ],
    )(_sc_body)
    return sc_fn(boxes_f, scores_f, labels_f, rank, maxc_f)


# speedup vs baseline: 145.7890x; 1.5714x over previous
import functools

import jax
import jax.numpy as jnp
from jax import lax
from jax.experimental import pallas as pl
from jax.experimental.pallas import tpu as pltpu
from jax.experimental.pallas import tpu_sc as plsc

_N = 5000
_NPAD = 5008
_NPAD2 = _NPAD + 16
_NKEY = 5120
_ROWS = _NKEY // 128
_SCORE_THRESH = 0.2
_NMS_THRESH = 0.5
_MAXK = 15


def _rank_body(srow_ref, scol_ref, bx_ref, rank_ref, maxc_ref):
    f32 = jnp.float32
    i32 = jnp.int32
    neg1 = f32(-1.0)
    thr = f32(_SCORE_THRESH)
    tri = (lax.broadcasted_iota(i32, (128, 128), 0)
           < lax.broadcasted_iota(i32, (128, 128), 1))

    for c in range(_ROWS):
        s_i = srow_ref[pl.ds(c, 1), :]
        ki = jnp.where(s_i >= thr, s_i, neg1)
        cnt = jnp.zeros((128,), i32)
        if c > 0:
            topv = scol_ref[pl.ds(0, c * 128), :]
            ktop = jnp.where(topv >= thr, topv, neg1)
            cnt = cnt + jnp.sum((ktop >= ki).astype(i32), axis=0)
        if c < _ROWS - 1:
            botv = scol_ref[pl.ds((c + 1) * 128, (_ROWS - 1 - c) * 128), :]
            kbot = jnp.where(botv >= thr, botv, neg1)
            cnt = cnt + jnp.sum((kbot > ki).astype(i32), axis=0)
        bandv = scol_ref[pl.ds(c * 128, 128), :]
        kband = jnp.where(bandv >= thr, bandv, neg1)
        tb = (kband > ki) | ((kband == ki) & tri)
        cnt = cnt + jnp.sum(tb.astype(i32), axis=0)
        rank_ref[pl.ds(c, 1), :] = cnt.reshape(1, 128)

    maxc_ref[...] = jnp.full((1, 128), jnp.max(bx_ref[...]), f32)


def _sc_body(boxes_hbm, scores_hbm, labels_hbm, rank_hbm, maxc_hbm, out_hbm,
             bx_v, sc_v, lb_v, rk_v, mx_v,
             sox0, soy0, sox1, soy1, sarea, ssc, slab,
             srx0, sry0, srx1, sry1,
             kox0, koy0, kox1, koy1, karea,
             orx0, ory0, orx1, ory1, osc, out_v):
    f32 = jnp.float32
    i32 = jnp.int32
    cid = lax.axis_index("c")
    sid = lax.axis_index("s")

    @pl.when((cid == 0) & (sid == 0))
    def _work():
        pltpu.sync_copy(boxes_hbm, bx_v)
        pltpu.sync_copy(scores_hbm, sc_v)
        pltpu.sync_copy(labels_hbm, lb_v)
        pltpu.sync_copy(rank_hbm, rk_v)
        pltpu.sync_copy(maxc_hbm, mx_v)

        m1 = lax.slice(mx_v[pl.ds(0, 16)], (0,), (1,))[0] + f32(1.0)

        lanes = lax.iota(i32, 16)

        def abody(k, _):
            idx = k * 16 + lanes
            r = rk_v[pl.ds(k * 16, 16)]
            s = sc_v[pl.ds(k * 16, 16)]
            l = lb_v[pl.ds(k * 16, 16)]
            x0 = plsc.load_gather(bx_v, [idx * 4])
            y0 = plsc.load_gather(bx_v, [idx * 4 + 1])
            x1 = plsc.load_gather(bx_v, [idx * 4 + 2])
            y1 = plsc.load_gather(bx_v, [idx * 4 + 3])
            off = l.astype(f32) * m1
            ox0 = x0 + off
            oy0 = y0 + off
            ox1 = x1 + off
            oy1 = y1 + off
            ar = (ox1 - ox0) * (oy1 - oy0)
            msk = idx < _N
            plsc.store_scatter(sox0, [r], ox0, mask=msk)
            plsc.store_scatter(soy0, [r], oy0, mask=msk)
            plsc.store_scatter(sox1, [r], ox1, mask=msk)
            plsc.store_scatter(soy1, [r], oy1, mask=msk)
            plsc.store_scatter(sarea, [r], ar, mask=msk)
            plsc.store_scatter(ssc, [r], s, mask=msk)
            plsc.store_scatter(slab, [r], l, mask=msk)
            plsc.store_scatter(srx0, [r], x0, mask=msk)
            plsc.store_scatter(sry0, [r], y0, mask=msk)
            plsc.store_scatter(srx1, [r], x1, mask=msk)
            plsc.store_scatter(sry1, [r], y1, mask=msk)
            return 0
        lax.fori_loop(0, _NPAD // 16, abody, 0)

        LO = jnp.full((16,), 1e30, f32)
        HI = jnp.full((16,), -1e30, f32)
        zf = jnp.zeros((16,), f32)
        for half in (0, 16):
            kox0[pl.ds(half, 16)] = LO
            koy0[pl.ds(half, 16)] = LO
            kox1[pl.ds(half, 16)] = HI
            koy1[pl.ds(half, 16)] = HI
            karea[pl.ds(half, 16)] = zf
            orx0[pl.ds(half, 16)] = zf
            ory0[pl.ds(half, 16)] = zf
            orx1[pl.ds(half, 16)] = zf
            ory1[pl.ds(half, 16)] = zf
            osc[pl.ds(half, 16)] = zf

        thr = f32(_SCORE_THRESH)

        def cond(carry):
            t, hc, oc, stop = carry
            return (~stop) & (t < _N)

        def body(carry):
            t, hc, oc, stop = carry
            civ = t + lanes
            s = plsc.load_gather(ssc, [civ])
            l = plsc.load_gather(slab, [civ])
            cox0 = plsc.load_gather(sox0, [civ])
            coy0 = plsc.load_gather(soy0, [civ])
            cox1 = plsc.load_gather(sox1, [civ])
            coy1 = plsc.load_gather(soy1, [civ])
            car = plsc.load_gather(sarea, [civ])
            ish = l == 0
            invalid = (s < thr) | (civ >= _N)
            skipv = jnp.where(ish, hc >= _MAXK, oc >= _MAXK)
            base = jnp.where(ish, 0, 16)
            sup = jnp.zeros((16,), jnp.bool_)
            for k in range(_MAXK):
                idxk = base + k
                kx0 = plsc.load_gather(kox0, [idxk])
                ky0 = plsc.load_gather(koy0, [idxk])
                kx1 = plsc.load_gather(kox1, [idxk])
                ky1 = plsc.load_gather(koy1, [idxk])
                kar = plsc.load_gather(karea, [idxk])
                lt0 = jnp.maximum(cox0, kx0)
                lt1 = jnp.maximum(coy0, ky0)
                rb0 = jnp.minimum(cox1, kx1)
                rb1 = jnp.minimum(coy1, ky1)
                w = jnp.maximum(rb0 - lt0, f32(0.0))
                h = jnp.maximum(rb1 - lt1, f32(0.0))
                inter = w * h
                union = car + kar - inter
                iou = inter / jnp.maximum(union, f32(1e-9))
                sup = sup | (iou > f32(_NMS_THRESH))
            appable = (~invalid) & (~skipv) & (~sup)
            fa = lax.slice(plsc.all_reduce_ffs(appable), (0,), (1,))[0]
            fi = lax.slice(plsc.all_reduce_ffs(invalid), (0,), (1,))[0]
            did_app = (fa < fi) & (fa < 16)
            hit_inv = (fi < 16) & (fi <= fa)
            ta = jnp.minimum(t + fa, i32(_N))

            tav = jnp.full((16,), ta, i32)
            la = plsc.load_gather(slab, [tav])
            ish_a = lax.slice(la, (0,), (1,))[0] == 0

            @pl.when(did_app)
            def _append():
                pos = jnp.where(ish_a, hc, oc + 16)
                pv = jnp.full((16,), pos, i32)
                one = lanes == 0
                plsc.store_scatter(kox0, [pv], plsc.load_gather(sox0, [tav]),
                                   mask=one)
                plsc.store_scatter(koy0, [pv], plsc.load_gather(soy0, [tav]),
                                   mask=one)
                plsc.store_scatter(kox1, [pv], plsc.load_gather(sox1, [tav]),
                                   mask=one)
                plsc.store_scatter(koy1, [pv], plsc.load_gather(soy1, [tav]),
                                   mask=one)
                plsc.store_scatter(karea, [pv], plsc.load_gather(sarea, [tav]),
                                   mask=one)
                plsc.store_scatter(orx0, [pv], plsc.load_gather(srx0, [tav]),
                                   mask=one)
                plsc.store_scatter(ory0, [pv], plsc.load_gather(sry0, [tav]),
                                   mask=one)
                plsc.store_scatter(orx1, [pv], plsc.load_gather(srx1, [tav]),
                                   mask=one)
                plsc.store_scatter(ory1, [pv], plsc.load_gather(sry1, [tav]),
                                   mask=one)
                plsc.store_scatter(osc, [pv], plsc.load_gather(ssc, [tav]),
                                   mask=one)

            inc = jnp.where(did_app, i32(1), i32(0))
            hc2 = hc + jnp.where(ish_a, inc, i32(0))
            oc2 = oc + jnp.where(ish_a, i32(0), inc)
            t2 = jnp.where(did_app, ta + 1, t + 16)
            stop2 = hit_inv | ((hc2 >= _MAXK) & (oc2 >= _MAXK))
            return (t2, hc2, oc2, stop2)

        lax.while_loop(cond, body, (i32(0), i32(0), i32(0), False))

        m15 = lanes < _MAXK
        for c, ref in enumerate([orx0, ory0, orx1, ory1, osc]):
            cv = jnp.full((16,), c, i32)
            plsc.store_scatter(out_v, [lanes, cv], ref[pl.ds(0, 16)],
                               mask=m15)
            plsc.store_scatter(out_v, [lanes + _MAXK, cv], ref[pl.ds(16, 16)],
                               mask=m15)
        pltpu.sync_copy(out_v, out_hbm)


def kernel(boxes, scores, labels):
    f32 = jnp.float32
    i32 = jnp.int32

    scores_p = jnp.pad(scores, (0, _NKEY - _N),
                       constant_values=jnp.float32(-1000.0))
    srow = scores_p.reshape(_ROWS, 128)
    scol = scores_p.reshape(_NKEY, 1)
    boxes_2d = jnp.pad(boxes.reshape(-1), (0, 20480 - 4 * _N)).reshape(160, 128)
    rank2d, maxc = pl.pallas_call(
        _rank_body,
        out_shape=(jax.ShapeDtypeStruct((_ROWS, 128), i32),
                   jax.ShapeDtypeStruct((1, 128), f32)),
    )(srow, scol, boxes_2d)
    rank = rank2d.reshape(_NKEY)[:_NPAD]
    maxc_f = maxc.reshape(128)

    boxes_f = jnp.pad(boxes.reshape(-1), (0, (_NPAD - _N) * 4))
    scores_f = jnp.pad(scores, (0, _NPAD - _N))
    labels_f = jnp.pad(labels, (0, _NPAD - _N))

    sc_fn = functools.partial(
        pl.kernel,
        out_type=jax.ShapeDtypeStruct((2 * _MAXK, 5), f32),
        mesh=plsc.VectorSubcoreMesh(core_axis_name="c", subcore_axis_name="s"),
        compiler_params=pltpu.CompilerParams(needs_layout_passes=False),
        scratch_types=[
            pltpu.VMEM((_NPAD * 4,), f32),
            pltpu.VMEM((_NPAD,), f32),
            pltpu.VMEM((_NPAD,), i32),
            pltpu.VMEM((_NPAD,), i32),
            pltpu.VMEM((128,), f32),
            pltpu.VMEM((_NPAD2,), f32),
            pltpu.VMEM((_NPAD2,), f32),
            pltpu.VMEM((_NPAD2,), f32),
            pltpu.VMEM((_NPAD2,), f32),
            pltpu.VMEM((_NPAD2,), f32),
            pltpu.VMEM((_NPAD2,), f32),
            pltpu.VMEM((_NPAD2,), i32),
            pltpu.VMEM((_NPAD2,), f32),
            pltpu.VMEM((_NPAD2,), f32),
            pltpu.VMEM((_NPAD2,), f32),
            pltpu.VMEM((_NPAD2,), f32),
            pltpu.VMEM((32,), f32),
            pltpu.VMEM((32,), f32),
            pltpu.VMEM((32,), f32),
            pltpu.VMEM((32,), f32),
            pltpu.VMEM((32,), f32),
            pltpu.VMEM((32,), f32),
            pltpu.VMEM((32,), f32),
            pltpu.VMEM((32,), f32),
            pltpu.VMEM((32,), f32),
            pltpu.VMEM((32,), f32),
            pltpu.VMEM((2 * _MAXK, 5), f32),
        ],
    )(_sc_body)
    return sc_fn(boxes_f, scores_f, labels_f, rank, maxc_f)


# speedup vs baseline: 150.2636x; 1.0307x over previous
import functools

import jax
import jax.numpy as jnp
from jax import lax
from jax.experimental import pallas as pl
from jax.experimental.pallas import tpu as pltpu
from jax.experimental.pallas import tpu_sc as plsc

_N = 5000
_NPAD = 5008
_NPAD2 = _NPAD + 16
_NKEY = 5120
_ROWS = _NKEY // 128
_SCORE_THRESH = 0.2
_NMS_THRESH = 0.5
_MAXK = 15


def _rank_body(srow_ref, scol_ref, rank_ref):
    f32 = jnp.float32
    i32 = jnp.int32
    neg1 = f32(-1.0)
    thr = f32(_SCORE_THRESH)
    tri = (lax.broadcasted_iota(i32, (128, 128), 0)
           < lax.broadcasted_iota(i32, (128, 128), 1))

    for c in range(_ROWS):
        s_i = srow_ref[pl.ds(c, 1), :]
        ki = jnp.where(s_i >= thr, s_i, neg1)
        cnt = jnp.zeros((128,), i32)
        if c > 0:
            topv = scol_ref[pl.ds(0, c * 128), :]
            ktop = jnp.where(topv >= thr, topv, neg1)
            cnt = cnt + jnp.sum((ktop >= ki).astype(i32), axis=0)
        if c < _ROWS - 1:
            botv = scol_ref[pl.ds((c + 1) * 128, (_ROWS - 1 - c) * 128), :]
            kbot = jnp.where(botv >= thr, botv, neg1)
            cnt = cnt + jnp.sum((kbot > ki).astype(i32), axis=0)
        bandv = scol_ref[pl.ds(c * 128, 128), :]
        kband = jnp.where(bandv >= thr, bandv, neg1)
        tb = (kband > ki) | ((kband == ki) & tri)
        cnt = cnt + jnp.sum(tb.astype(i32), axis=0)
        rank_ref[pl.ds(c, 1), :] = cnt.reshape(1, 128)


def _sc_body(boxes_hbm, scores_hbm, labels_hbm, rank_hbm, out_hbm,
             bx_v, sc_v, lb_v, rk_v,
             sox0, soy0, sox1, soy1, sarea, ssc, slab,
             srx0, sry0, srx1, sry1,
             kox0, koy0, kox1, koy1, karea,
             orx0, ory0, orx1, ory1, osc, out_v):
    f32 = jnp.float32
    i32 = jnp.int32
    cid = lax.axis_index("c")
    sid = lax.axis_index("s")

    @pl.when((cid == 0) & (sid == 0))
    def _work():
        pltpu.sync_copy(boxes_hbm, bx_v)
        pltpu.sync_copy(scores_hbm, sc_v)
        pltpu.sync_copy(labels_hbm, lb_v)
        pltpu.sync_copy(rank_hbm, rk_v)

        lanes = lax.iota(i32, 16)

        def a1body(k, mx):
            idx = k * 16 + lanes
            idxc = jnp.minimum(idx, i32(_N - 1))
            r = rk_v[pl.ds(k * 16, 16)]
            s = sc_v[pl.ds(k * 16, 16)]
            l = lb_v[pl.ds(k * 16, 16)]
            x0 = plsc.load_gather(bx_v, [idxc * 4])
            y0 = plsc.load_gather(bx_v, [idxc * 4 + 1])
            x1 = plsc.load_gather(bx_v, [idxc * 4 + 2])
            y1 = plsc.load_gather(bx_v, [idxc * 4 + 3])
            msk = idx < _N
            plsc.store_scatter(ssc, [r], s, mask=msk)
            plsc.store_scatter(slab, [r], l, mask=msk)
            plsc.store_scatter(srx0, [r], x0, mask=msk)
            plsc.store_scatter(sry0, [r], y0, mask=msk)
            plsc.store_scatter(srx1, [r], x1, mask=msk)
            plsc.store_scatter(sry1, [r], y1, mask=msk)
            return jnp.maximum(mx, jnp.maximum(x1, y1))
        mvec = lax.fori_loop(0, _NPAD // 16, a1body,
                             jnp.full((16,), -3e38, f32))
        m1 = jnp.max(mvec) + f32(1.0)

        def a2body(j, _):
            b = j * 16
            rx0 = srx0[pl.ds(b, 16)]
            ry0 = sry0[pl.ds(b, 16)]
            rx1 = srx1[pl.ds(b, 16)]
            ry1 = sry1[pl.ds(b, 16)]
            l = slab[pl.ds(b, 16)]
            off = l.astype(f32) * m1
            ox0 = rx0 + off
            oy0 = ry0 + off
            ox1 = rx1 + off
            oy1 = ry1 + off
            sox0[pl.ds(b, 16)] = ox0
            soy0[pl.ds(b, 16)] = oy0
            sox1[pl.ds(b, 16)] = ox1
            soy1[pl.ds(b, 16)] = oy1
            sarea[pl.ds(b, 16)] = (ox1 - ox0) * (oy1 - oy0)
            return 0
        lax.fori_loop(0, _NPAD // 16, a2body, 0)

        LO = jnp.full((16,), 1e30, f32)
        HI = jnp.full((16,), -1e30, f32)
        zf = jnp.zeros((16,), f32)
        for half in (0, 16):
            kox0[pl.ds(half, 16)] = LO
            koy0[pl.ds(half, 16)] = LO
            kox1[pl.ds(half, 16)] = HI
            koy1[pl.ds(half, 16)] = HI
            karea[pl.ds(half, 16)] = zf
            orx0[pl.ds(half, 16)] = zf
            ory0[pl.ds(half, 16)] = zf
            orx1[pl.ds(half, 16)] = zf
            ory1[pl.ds(half, 16)] = zf
            osc[pl.ds(half, 16)] = zf

        thr = f32(_SCORE_THRESH)

        def cond(carry):
            t, hc, oc, stop = carry
            return (~stop) & (t < _N)

        def body(carry):
            t, hc, oc, stop = carry
            civ = t + lanes
            s = plsc.load_gather(ssc, [civ])
            l = plsc.load_gather(slab, [civ])
            cox0 = plsc.load_gather(sox0, [civ])
            coy0 = plsc.load_gather(soy0, [civ])
            cox1 = plsc.load_gather(sox1, [civ])
            coy1 = plsc.load_gather(soy1, [civ])
            car = plsc.load_gather(sarea, [civ])
            ish = l == 0
            invalid = (s < thr) | (civ >= _N)
            skipv = jnp.where(ish, hc >= _MAXK, oc >= _MAXK)
            base = jnp.where(ish, 0, 16)
            sup = jnp.zeros((16,), jnp.bool_)
            for k in range(_MAXK):
                idxk = base + k
                kx0 = plsc.load_gather(kox0, [idxk])
                ky0 = plsc.load_gather(koy0, [idxk])
                kx1 = plsc.load_gather(kox1, [idxk])
                ky1 = plsc.load_gather(koy1, [idxk])
                kar = plsc.load_gather(karea, [idxk])
                lt0 = jnp.maximum(cox0, kx0)
                lt1 = jnp.maximum(coy0, ky0)
                rb0 = jnp.minimum(cox1, kx1)
                rb1 = jnp.minimum(coy1, ky1)
                w = jnp.maximum(rb0 - lt0, f32(0.0))
                h = jnp.maximum(rb1 - lt1, f32(0.0))
                inter = w * h
                union = car + kar - inter
                iou = inter / jnp.maximum(union, f32(1e-9))
                sup = sup | (iou > f32(_NMS_THRESH))
            appable = (~invalid) & (~skipv) & (~sup)
            fa = lax.slice(plsc.all_reduce_ffs(appable), (0,), (1,))[0]
            fi = lax.slice(plsc.all_reduce_ffs(invalid), (0,), (1,))[0]
            did_app = (fa < fi) & (fa < 16)
            hit_inv = (fi < 16) & (fi <= fa)
            ta = jnp.minimum(t + fa, i32(_N))

            tav = jnp.full((16,), ta, i32)
            la = plsc.load_gather(slab, [tav])
            ish_a = lax.slice(la, (0,), (1,))[0] == 0

            @pl.when(did_app)
            def _append():
                pos = jnp.where(ish_a, hc, oc + 16)
                pv = jnp.full((16,), pos, i32)
                one = lanes == 0
                plsc.store_scatter(kox0, [pv], plsc.load_gather(sox0, [tav]),
                                   mask=one)
                plsc.store_scatter(koy0, [pv], plsc.load_gather(soy0, [tav]),
                                   mask=one)
                plsc.store_scatter(kox1, [pv], plsc.load_gather(sox1, [tav]),
                                   mask=one)
                plsc.store_scatter(koy1, [pv], plsc.load_gather(soy1, [tav]),
                                   mask=one)
                plsc.store_scatter(karea, [pv], plsc.load_gather(sarea, [tav]),
                                   mask=one)
                plsc.store_scatter(orx0, [pv], plsc.load_gather(srx0, [tav]),
                                   mask=one)
                plsc.store_scatter(ory0, [pv], plsc.load_gather(sry0, [tav]),
                                   mask=one)
                plsc.store_scatter(orx1, [pv], plsc.load_gather(srx1, [tav]),
                                   mask=one)
                plsc.store_scatter(ory1, [pv], plsc.load_gather(sry1, [tav]),
                                   mask=one)
                plsc.store_scatter(osc, [pv], plsc.load_gather(ssc, [tav]),
                                   mask=one)

            inc = jnp.where(did_app, i32(1), i32(0))
            hc2 = hc + jnp.where(ish_a, inc, i32(0))
            oc2 = oc + jnp.where(ish_a, i32(0), inc)
            t2 = jnp.where(did_app, ta + 1, t + 16)
            stop2 = hit_inv | ((hc2 >= _MAXK) & (oc2 >= _MAXK))
            return (t2, hc2, oc2, stop2)

        lax.while_loop(cond, body, (i32(0), i32(0), i32(0), False))

        m15 = lanes < _MAXK
        for c, ref in enumerate([orx0, ory0, orx1, ory1, osc]):
            cv = jnp.full((16,), c, i32)
            plsc.store_scatter(out_v, [lanes, cv], ref[pl.ds(0, 16)],
                               mask=m15)
            plsc.store_scatter(out_v, [lanes + _MAXK, cv], ref[pl.ds(16, 16)],
                               mask=m15)
        pltpu.sync_copy(out_v, out_hbm)


def kernel(boxes, scores, labels):
    f32 = jnp.float32
    i32 = jnp.int32

    scores_p = jnp.pad(scores, (0, _NKEY - _N),
                       constant_values=jnp.float32(-1000.0))
    srow = scores_p.reshape(_ROWS, 128)
    scol = scores_p.reshape(_NKEY, 1)
    rank2d = pl.pallas_call(
        _rank_body,
        out_shape=jax.ShapeDtypeStruct((_ROWS, 128), i32),
    )(srow, scol)
    rank = rank2d.reshape(_NKEY)[:_NPAD]

    scores_f = scores_p[:_NPAD]
    labels_f = jnp.pad(labels, (0, _NPAD - _N))

    sc_fn = functools.partial(
        pl.kernel,
        out_type=jax.ShapeDtypeStruct((2 * _MAXK, 5), f32),
        mesh=plsc.VectorSubcoreMesh(core_axis_name="c", subcore_axis_name="s"),
        compiler_params=pltpu.CompilerParams(needs_layout_passes=False),
        scratch_types=[
            pltpu.VMEM((_N * 4,), f32),
            pltpu.VMEM((_NPAD,), f32),
            pltpu.VMEM((_NPAD,), i32),
            pltpu.VMEM((_NPAD,), i32),
            pltpu.VMEM((_NPAD2,), f32),
            pltpu.VMEM((_NPAD2,), f32),
            pltpu.VMEM((_NPAD2,), f32),
            pltpu.VMEM((_NPAD2,), f32),
            pltpu.VMEM((_NPAD2,), f32),
            pltpu.VMEM((_NPAD2,), f32),
            pltpu.VMEM((_NPAD2,), i32),
            pltpu.VMEM((_NPAD2,), f32),
            pltpu.VMEM((_NPAD2,), f32),
            pltpu.VMEM((_NPAD2,), f32),
            pltpu.VMEM((_NPAD2,), f32),
            pltpu.VMEM((32,), f32),
            pltpu.VMEM((32,), f32),
            pltpu.VMEM((32,), f32),
            pltpu.VMEM((32,), f32),
            pltpu.VMEM((32,), f32),
            pltpu.VMEM((32,), f32),
            pltpu.VMEM((32,), f32),
            pltpu.VMEM((32,), f32),
            pltpu.VMEM((32,), f32),
            pltpu.VMEM((32,), f32),
            pltpu.VMEM((2 * _MAXK, 5), f32),
        ],
    )(_sc_body)
    return sc_fn(boxes.reshape(-1), scores_f, labels_f, rank)


# speedup vs baseline: 155.8869x; 1.0374x over previous
import functools

import jax
import jax.numpy as jnp
from jax import lax
from jax.experimental import pallas as pl
from jax.experimental.pallas import tpu as pltpu
from jax.experimental.pallas import tpu_sc as plsc

_N = 5000
_NPAD = 5008
_NPAD2 = _NPAD + 16
_NKEY = 5120
_ROWS = _NKEY // 128
_SCORE_THRESH = 0.2
_NMS_THRESH = 0.5
_MAXK = 15


def _rank_body(srow_ref, scol_ref, rank_ref):
    f32 = jnp.float32
    i32 = jnp.int32
    neg1 = f32(-1.0)
    thr = f32(_SCORE_THRESH)
    tri = (lax.broadcasted_iota(i32, (128, 128), 0)
           < lax.broadcasted_iota(i32, (128, 128), 1))

    for c in range(_ROWS):
        s_i = srow_ref[pl.ds(c, 1), :]
        ki = jnp.where(s_i >= thr, s_i, neg1)
        cnt = jnp.zeros((128,), i32)
        if c > 0:
            topv = scol_ref[pl.ds(0, c * 128), :]
            ktop = jnp.where(topv >= thr, topv, neg1)
            cnt = cnt + jnp.sum((ktop >= ki).astype(i32), axis=0)
        if c < _ROWS - 1:
            botv = scol_ref[pl.ds((c + 1) * 128, (_ROWS - 1 - c) * 128), :]
            kbot = jnp.where(botv >= thr, botv, neg1)
            cnt = cnt + jnp.sum((kbot > ki).astype(i32), axis=0)
        bandv = scol_ref[pl.ds(c * 128, 128), :]
        kband = jnp.where(bandv >= thr, bandv, neg1)
        tb = (kband > ki) | ((kband == ki) & tri)
        cnt = cnt + jnp.sum(tb.astype(i32), axis=0)
        rank_ref[pl.ds(c, 1), :] = cnt.reshape(1, 128)


def _sc_body(boxes_hbm, scores_hbm, labels_hbm, rank_hbm, out_hbm,
             bx_v, sc_v, lb_v, rk_v,
             sox0, soy0, sox1, soy1, sarea, ssc, slab,
             srx0, sry0, srx1, sry1,
             kox0, koy0, kox1, koy1, karea,
             orx0, ory0, orx1, ory1, osc, out_v):
    f32 = jnp.float32
    i32 = jnp.int32
    cid = lax.axis_index("c")
    sid = lax.axis_index("s")

    @pl.when((cid == 0) & (sid == 0))
    def _work():
        pltpu.sync_copy(boxes_hbm, bx_v)
        pltpu.sync_copy(scores_hbm.at[pl.ds(0, _NPAD)], sc_v)
        pltpu.sync_copy(labels_hbm, lb_v)
        pltpu.sync_copy(rank_hbm, rk_v)

        lanes = lax.iota(i32, 16)

        def a1body(k, mx):
            idx = k * 16 + lanes
            idxc = jnp.minimum(idx, i32(_N - 1))
            r = plsc.load_gather(
                rk_v, [jnp.full((16,), k // 8, i32), (k % 8) * 16 + lanes])
            s = sc_v[pl.ds(k * 16, 16)]
            l = lb_v[pl.ds(k * 16, 16)]
            x0 = plsc.load_gather(bx_v, [idxc * 4])
            y0 = plsc.load_gather(bx_v, [idxc * 4 + 1])
            x1 = plsc.load_gather(bx_v, [idxc * 4 + 2])
            y1 = plsc.load_gather(bx_v, [idxc * 4 + 3])
            msk = idx < _N
            plsc.store_scatter(ssc, [r], s, mask=msk)
            plsc.store_scatter(slab, [r], l, mask=msk)
            plsc.store_scatter(srx0, [r], x0, mask=msk)
            plsc.store_scatter(sry0, [r], y0, mask=msk)
            plsc.store_scatter(srx1, [r], x1, mask=msk)
            plsc.store_scatter(sry1, [r], y1, mask=msk)
            return jnp.maximum(mx, jnp.maximum(x1, y1))
        mvec = lax.fori_loop(0, _NPAD // 16, a1body,
                             jnp.full((16,), -3e38, f32))
        m1 = jnp.max(mvec) + f32(1.0)

        def a2body(j, _):
            b = j * 16
            rx0 = srx0[pl.ds(b, 16)]
            ry0 = sry0[pl.ds(b, 16)]
            rx1 = srx1[pl.ds(b, 16)]
            ry1 = sry1[pl.ds(b, 16)]
            l = slab[pl.ds(b, 16)]
            off = l.astype(f32) * m1
            ox0 = rx0 + off
            oy0 = ry0 + off
            ox1 = rx1 + off
            oy1 = ry1 + off
            sox0[pl.ds(b, 16)] = ox0
            soy0[pl.ds(b, 16)] = oy0
            sox1[pl.ds(b, 16)] = ox1
            soy1[pl.ds(b, 16)] = oy1
            sarea[pl.ds(b, 16)] = (ox1 - ox0) * (oy1 - oy0)
            return 0
        lax.fori_loop(0, _NPAD // 16, a2body, 0)

        LO = jnp.full((16,), 1e30, f32)
        HI = jnp.full((16,), -1e30, f32)
        zf = jnp.zeros((16,), f32)
        for half in (0, 16):
            kox0[pl.ds(half, 16)] = LO
            koy0[pl.ds(half, 16)] = LO
            kox1[pl.ds(half, 16)] = HI
            koy1[pl.ds(half, 16)] = HI
            karea[pl.ds(half, 16)] = zf
            orx0[pl.ds(half, 16)] = zf
            ory0[pl.ds(half, 16)] = zf
            orx1[pl.ds(half, 16)] = zf
            ory1[pl.ds(half, 16)] = zf
            osc[pl.ds(half, 16)] = zf

        thr = f32(_SCORE_THRESH)

        def cond(carry):
            t, hc, oc, stop = carry
            return (~stop) & (t < _N)

        def body(carry):
            t, hc, oc, stop = carry
            civ = t + lanes
            s = plsc.load_gather(ssc, [civ])
            l = plsc.load_gather(slab, [civ])
            cox0 = plsc.load_gather(sox0, [civ])
            coy0 = plsc.load_gather(soy0, [civ])
            cox1 = plsc.load_gather(sox1, [civ])
            coy1 = plsc.load_gather(soy1, [civ])
            car = plsc.load_gather(sarea, [civ])
            ish = l == 0
            invalid = (s < thr) | (civ >= _N)
            skipv = jnp.where(ish, hc >= _MAXK, oc >= _MAXK)
            base = jnp.where(ish, 0, 16)
            maybe = (~invalid) & (~skipv)

            def _slot_loop():
                sup = jnp.zeros((16,), jnp.bool_)
                for k in range(_MAXK):
                    idxk = base + k
                    kx0 = plsc.load_gather(kox0, [idxk])
                    ky0 = plsc.load_gather(koy0, [idxk])
                    kx1 = plsc.load_gather(kox1, [idxk])
                    ky1 = plsc.load_gather(koy1, [idxk])
                    kar = plsc.load_gather(karea, [idxk])
                    lt0 = jnp.maximum(cox0, kx0)
                    lt1 = jnp.maximum(coy0, ky0)
                    rb0 = jnp.minimum(cox1, kx1)
                    rb1 = jnp.minimum(coy1, ky1)
                    w = jnp.maximum(rb0 - lt0, f32(0.0))
                    h = jnp.maximum(rb1 - lt1, f32(0.0))
                    inter = w * h
                    union = car + kar - inter
                    iou = inter / jnp.maximum(union, f32(1e-9))
                    sup = sup | (iou > f32(_NMS_THRESH))
                return sup

            sup = lax.cond(jnp.any(maybe), _slot_loop,
                           lambda: jnp.zeros((16,), jnp.bool_))
            appable = maybe & (~sup)
            fa = lax.slice(plsc.all_reduce_ffs(appable), (0,), (1,))[0]
            fi = lax.slice(plsc.all_reduce_ffs(invalid), (0,), (1,))[0]
            did_app = (fa < fi) & (fa < 16)
            hit_inv = (fi < 16) & (fi <= fa)
            ta = jnp.minimum(t + fa, i32(_N))

            tav = jnp.full((16,), ta, i32)
            la = plsc.load_gather(slab, [tav])
            ish_a = lax.slice(la, (0,), (1,))[0] == 0

            @pl.when(did_app)
            def _append():
                pos = jnp.where(ish_a, hc, oc + 16)
                pv = jnp.full((16,), pos, i32)
                one = lanes == 0
                plsc.store_scatter(kox0, [pv], plsc.load_gather(sox0, [tav]),
                                   mask=one)
                plsc.store_scatter(koy0, [pv], plsc.load_gather(soy0, [tav]),
                                   mask=one)
                plsc.store_scatter(kox1, [pv], plsc.load_gather(sox1, [tav]),
                                   mask=one)
                plsc.store_scatter(koy1, [pv], plsc.load_gather(soy1, [tav]),
                                   mask=one)
                plsc.store_scatter(karea, [pv], plsc.load_gather(sarea, [tav]),
                                   mask=one)
                plsc.store_scatter(orx0, [pv], plsc.load_gather(srx0, [tav]),
                                   mask=one)
                plsc.store_scatter(ory0, [pv], plsc.load_gather(sry0, [tav]),
                                   mask=one)
                plsc.store_scatter(orx1, [pv], plsc.load_gather(srx1, [tav]),
                                   mask=one)
                plsc.store_scatter(ory1, [pv], plsc.load_gather(sry1, [tav]),
                                   mask=one)
                plsc.store_scatter(osc, [pv], plsc.load_gather(ssc, [tav]),
                                   mask=one)

            inc = jnp.where(did_app, i32(1), i32(0))
            hc2 = hc + jnp.where(ish_a, inc, i32(0))
            oc2 = oc + jnp.where(ish_a, i32(0), inc)
            t2 = jnp.where(did_app, ta + 1, t + 16)
            stop2 = hit_inv | ((hc2 >= _MAXK) & (oc2 >= _MAXK))
            return (t2, hc2, oc2, stop2)

        lax.while_loop(cond, body, (i32(0), i32(0), i32(0), False))

        m15 = lanes < _MAXK
        for c, ref in enumerate([orx0, ory0, orx1, ory1, osc]):
            cv = jnp.full((16,), c, i32)
            plsc.store_scatter(out_v, [lanes, cv], ref[pl.ds(0, 16)],
                               mask=m15)
            plsc.store_scatter(out_v, [lanes + _MAXK, cv], ref[pl.ds(16, 16)],
                               mask=m15)
        pltpu.sync_copy(out_v, out_hbm)


def kernel(boxes, scores, labels):
    f32 = jnp.float32
    i32 = jnp.int32

    scores_p = jnp.pad(scores, (0, _NKEY - _N),
                       constant_values=jnp.float32(-1000.0))
    srow = scores_p.reshape(_ROWS, 128)
    scol = scores_p.reshape(_NKEY, 1)
    rank2d = pl.pallas_call(
        _rank_body,
        out_shape=jax.ShapeDtypeStruct((_ROWS, 128), i32),
    )(srow, scol)
    labels_f = jnp.pad(labels, (0, _NPAD - _N))

    sc_fn = functools.partial(
        pl.kernel,
        out_type=jax.ShapeDtypeStruct((2 * _MAXK, 5), f32),
        mesh=plsc.VectorSubcoreMesh(core_axis_name="c", subcore_axis_name="s"),
        compiler_params=pltpu.CompilerParams(needs_layout_passes=False),
        scratch_types=[
            pltpu.VMEM((_N * 4,), f32),
            pltpu.VMEM((_NPAD,), f32),
            pltpu.VMEM((_NPAD,), i32),
            pltpu.VMEM((_ROWS, 128), i32),
            pltpu.VMEM((_NPAD2,), f32),
            pltpu.VMEM((_NPAD2,), f32),
            pltpu.VMEM((_NPAD2,), f32),
            pltpu.VMEM((_NPAD2,), f32),
            pltpu.VMEM((_NPAD2,), f32),
            pltpu.VMEM((_NPAD2,), f32),
            pltpu.VMEM((_NPAD2,), i32),
            pltpu.VMEM((_NPAD2,), f32),
            pltpu.VMEM((_NPAD2,), f32),
            pltpu.VMEM((_NPAD2,), f32),
            pltpu.VMEM((_NPAD2,), f32),
            pltpu.VMEM((32,), f32),
            pltpu.VMEM((32,), f32),
            pltpu.VMEM((32,), f32),
            pltpu.VMEM((32,), f32),
            pltpu.VMEM((32,), f32),
            pltpu.VMEM((32,), f32),
            pltpu.VMEM((32,), f32),
            pltpu.VMEM((32,), f32),
            pltpu.VMEM((32,), f32),
            pltpu.VMEM((32,), f32),
            pltpu.VMEM((2 * _MAXK, 5), f32),
        ],
    )(_sc_body)
    return sc_fn(boxes.reshape(-1), scores_p, labels_f, rank2d)
